# baseline (speedup 1.0000x reference)
import functools

import jax
import jax.numpy as jnp
from jax import lax
from jax.experimental import pallas as pl
from jax.experimental.pallas import tpu as pltpu
from jax.experimental.pallas import tpu_sc as plsc

HIDDEN_DIM = 64
PAIR_DIM = 128
GAMMA = 12.0
BATCH = 16384

_INFO = plsc.get_sparse_core_info()
_NC = _INFO.num_cores
_NS = _INFO.num_subcores
_NW = _NC * _NS
_BPW = BATCH // _NW
_HALF = _BPW // 2
_CHUNK = 128
_NCHUNK = _BPW // _CHUNK
_GROUPS = _HALF // 16


def _make_kernel():
    mesh = plsc.VectorSubcoreMesh(core_axis_name="c", subcore_axis_name="s")

    @functools.partial(
        pl.kernel,
        mesh=mesh,
        out_type=jax.ShapeDtypeStruct((BATCH,), jnp.float32),
        scratch_types=[
            pltpu.VMEM((_NCHUNK, 1, _CHUNK), jnp.int32),
            pltpu.VMEM((_NCHUNK, 1, _CHUNK), jnp.int32),
            pltpu.VMEM((_NCHUNK, 1, _CHUNK), jnp.int32),
            pltpu.VMEM((_NCHUNK, 1, _CHUNK), jnp.int32),
            pltpu.VMEM((_NCHUNK, 1, _CHUNK), jnp.int32),
            pltpu.VMEM((_NCHUNK, 1, _CHUNK), jnp.int32),
            pltpu.VMEM((_HALF, PAIR_DIM), jnp.float32),
            pltpu.VMEM((_HALF, PAIR_DIM), jnp.float32),
            pltpu.VMEM((_HALF, PAIR_DIM), jnp.float32),
            pltpu.VMEM((_BPW,), jnp.float32),
            pltpu.SemaphoreType.DMA,
        ],
        compiler_params=pltpu.CompilerParams(
            needs_layout_passes=False, use_tc_tiling_on_sc=True),
    )
    def kge_score(h_hbm, r_hbm, t_hbm, ent_hbm, rel_hbm, out_hbm,
                  idx_h, idx_r, idx_t, pidx_h, pidx_r, pidx_t,
                  rows_h, rows_r, rows_t, out_v, sem):
        wid = lax.axis_index("s") * _NC + lax.axis_index("c")
        base = wid * _BPW

        stage = []
        for j in range(_NCHUNK):
            src = pl.ds(base + j * _CHUNK, _CHUNK)
            stage.append(pltpu.async_copy(h_hbm.at[src], idx_h.at[j, 0], sem))
            stage.append(pltpu.async_copy(r_hbm.at[src], idx_r.at[j, 0], sem))
            stage.append(pltpu.async_copy(t_hbm.at[src], idx_t.at[j, 0], sem))
        for c in stage:
            c.wait()

        for j in range(_NCHUNK):
            for k in range(_CHUNK // 16):
                s = pl.ds(k * 16, 16)
                pidx_h[j, 0, s] = idx_h[j, 0, s] >> 1
                pidx_r[j, 0, s] = idx_r[j, 0, s] >> 1
                pidx_t[j, 0, s] = idx_t[j, 0, s] >> 1

        lane = lax.iota(jnp.int32, 16)

        for half in range(2):
            copies = []
            for j in range(2):
                chunk = half * 2 + j
                dst = pl.ds(j * _CHUNK, _CHUNK)
                copies.append(pltpu.async_copy(
                    ent_hbm.at[pidx_h.at[chunk, 0]], rows_h.at[dst], sem))
                copies.append(pltpu.async_copy(
                    rel_hbm.at[pidx_r.at[chunk, 0]], rows_r.at[dst], sem))
                copies.append(pltpu.async_copy(
                    ent_hbm.at[pidx_t.at[chunk, 0]], rows_t.at[dst], sem))
            for c in copies:
                c.wait()

            def group_body(g, carry):
                rids = g * 16 + lane
                chunk = half * 2 + g // 8
                s = pl.ds((g % 8) * 16, 16)
                off_h = (idx_h[chunk, 0, s] & 1) * HIDDEN_DIM
                off_r = (idx_r[chunk, 0, s] & 1) * HIDDEN_DIM
                off_t = (idx_t[chunk, 0, s] & 1) * HIDDEN_DIM
                acc = jnp.zeros((16,), jnp.float32)
                for d in range(HIDDEN_DIM):
                    hv = plsc.load_gather(rows_h, [rids, off_h + d])
                    rv = plsc.load_gather(rows_r, [rids, off_r + d])
                    tv = plsc.load_gather(rows_t, [rids, off_t + d])
                    acc = acc + jnp.abs(hv + rv - tv)
                out_v[pl.ds(half * _HALF + g * 16, 16)] = GAMMA - acc
                return carry

            lax.fori_loop(0, _GROUPS, group_body, 0)

        pltpu.sync_copy(out_v, out_hbm.at[pl.ds(base, _BPW)])

    return kge_score


_KERNEL = _make_kernel()


def kernel(sample, entity_embedding, relation_embedding):
    ent_p = entity_embedding.reshape(500000, PAIR_DIM)
    rel_p = relation_embedding.reshape(500000, PAIR_DIM)
    scores = _KERNEL(sample[:, 0], sample[:, 1], sample[:, 2], ent_p, rel_p)
    return scores.reshape(BATCH, 1)


# speedup vs baseline: 1.4116x; 1.4116x over previous
import functools

import jax
import jax.numpy as jnp
from jax import lax
from jax.experimental import pallas as pl
from jax.experimental.pallas import tpu as pltpu
from jax.experimental.pallas import tpu_sc as plsc

HIDDEN_DIM = 64
GAMMA = 12.0
BATCH = 16384

_INFO = plsc.get_sparse_core_info()
_NC = _INFO.num_cores
_NS = _INFO.num_subcores
_NW = _NC * _NS
_BPW = BATCH // _NW
_CHUNK = 128
_NCHUNK = _BPW // _CHUNK
_TPC = 32
_NPH = _BPW // _TPC


def _make_kernel():
    mesh = plsc.VectorSubcoreMesh(core_axis_name="c", subcore_axis_name="s")

    @functools.partial(
        pl.kernel,
        mesh=mesh,
        out_type=jax.ShapeDtypeStruct((BATCH,), jnp.float32),
        scratch_types=[
            pltpu.VMEM((_NCHUNK, 1, _CHUNK), jnp.int32),
            pltpu.VMEM((_NCHUNK, 1, _CHUNK), jnp.int32),
            pltpu.VMEM((_NCHUNK, 1, _CHUNK), jnp.int32),
            pltpu.VMEM((_TPC, 8, HIDDEN_DIM), jnp.float32),
            pltpu.VMEM((_TPC, 8, HIDDEN_DIM), jnp.float32),
            pltpu.VMEM((_TPC, 8, HIDDEN_DIM), jnp.float32),
            pltpu.VMEM((_BPW,), jnp.float32),
            pltpu.SemaphoreType.DMA,
        ],
        compiler_params=pltpu.CompilerParams(
            needs_layout_passes=False, use_tc_tiling_on_sc=True),
    )
    def kge_score(h_hbm, r_hbm, t_hbm, ent_hbm, rel_hbm, out_hbm,
                  idx_h, idx_r, idx_t, rows_h, rows_r, rows_t, out_v, sem):
        wid = lax.axis_index("s") * _NC + lax.axis_index("c")
        base = wid * _BPW

        stage = []
        for j in range(_NCHUNK):
            src = pl.ds(base + j * _CHUNK, _CHUNK)
            stage.append(pltpu.async_copy(h_hbm.at[src], idx_h.at[j, 0], sem))
            stage.append(pltpu.async_copy(r_hbm.at[src], idx_r.at[j, 0], sem))
            stage.append(pltpu.async_copy(t_hbm.at[src], idx_t.at[j, 0], sem))
        for c in stage:
            c.wait()

        lane = lax.iota(jnp.int32, 16)

        def chunk_body(c, carry):
            j = c // 4
            off = (c % 4) * _TPC

            vecs = []
            for g in range(2):
                s = pl.ds(off + g * 16, 16)
                vecs.append((idx_h[j, 0, s], idx_r[j, 0, s], idx_t[j, 0, s]))

            copies = []
            for g in range(2):
                vh, vr, vt = vecs[g]
                for k in range(16):
                    slot = g * 16 + k
                    for v, tbl, dst in ((vh, ent_hbm, rows_h),
                                        (vr, rel_hbm, rows_r),
                                        (vt, ent_hbm, rows_t)):
                        r0 = pl.multiple_of((v[k] >> 3) * 8, 8)
                        copies.append(pltpu.async_copy(
                            tbl.at[pl.ds(r0, 8), :], dst.at[slot], sem))
            for cp in copies:
                cp.wait()

            for g in range(2):
                vh, vr, vt = vecs[g]
                slotv = g * 16 + lane
                mh, mr, mt = vh & 7, vr & 7, vt & 7
                acc = jnp.zeros((16,), jnp.float32)
                for d in range(HIDDEN_DIM):
                    dv = jnp.full((16,), d, jnp.int32)
                    hv = plsc.load_gather(rows_h, [slotv, mh, dv])
                    rv = plsc.load_gather(rows_r, [slotv, mr, dv])
                    tv = plsc.load_gather(rows_t, [slotv, mt, dv])
                    acc = acc + jnp.abs(hv + rv - tv)
                out_v[pl.ds(c * _TPC + g * 16, 16)] = GAMMA - acc
            return carry

        lax.fori_loop(0, _NPH, chunk_body, 0)

        pltpu.sync_copy(out_v, out_hbm.at[pl.ds(base, _BPW)])

    return kge_score


_KERNEL = _make_kernel()


def kernel(sample, entity_embedding, relation_embedding):
    scores = _KERNEL(sample[:, 0], sample[:, 1], sample[:, 2],
                     entity_embedding, relation_embedding)
    return scores.reshape(BATCH, 1)


# speedup vs baseline: 2.0294x; 1.4377x over previous
import functools

import jax
import jax.numpy as jnp
from jax import lax
from jax.experimental import pallas as pl
from jax.experimental.pallas import tpu as pltpu
from jax.experimental.pallas import tpu_sc as plsc

HIDDEN_DIM = 64
GAMMA = 12.0
BATCH = 16384

_INFO = plsc.get_sparse_core_info()
_NC = _INFO.num_cores
_NS = _INFO.num_subcores
_NW = _NC * _NS
_BPW = BATCH // _NW
_CHUNK = 128
_NCHUNK = _BPW // _CHUNK
_TPC = 32
_NPH = _BPW // _TPC


def _make_kernel():
    mesh = plsc.VectorSubcoreMesh(core_axis_name="c", subcore_axis_name="s")

    @functools.partial(
        pl.kernel,
        mesh=mesh,
        out_type=jax.ShapeDtypeStruct((BATCH,), jnp.float32),
        scratch_types=[
            pltpu.VMEM((_NCHUNK, 1, _CHUNK), jnp.int32),
            pltpu.VMEM((_NCHUNK, 1, _CHUNK), jnp.int32),
            pltpu.VMEM((_NCHUNK, 1, _CHUNK), jnp.int32),
            pltpu.VMEM((_TPC, 8, HIDDEN_DIM), jnp.float32),
            pltpu.VMEM((_TPC, 8, HIDDEN_DIM), jnp.float32),
            pltpu.VMEM((_TPC, 8, HIDDEN_DIM), jnp.float32),
            pltpu.VMEM((_BPW,), jnp.float32),
            pltpu.SemaphoreType.DMA,
        ],
        compiler_params=pltpu.CompilerParams(
            needs_layout_passes=False, use_tc_tiling_on_sc=True),
    )
    def kge_score(h_hbm, r_hbm, t_hbm, ent_hbm, rel_hbm, out_hbm,
                  idx_h, idx_r, idx_t, rows_h, rows_r, rows_t, out_v, sem):
        wid = lax.axis_index("s") * _NC + lax.axis_index("c")
        base = wid * _BPW

        stage = []
        for j in range(_NCHUNK):
            src = pl.ds(base + j * _CHUNK, _CHUNK)
            stage.append(pltpu.async_copy(h_hbm.at[src], idx_h.at[j, 0], sem))
            stage.append(pltpu.async_copy(r_hbm.at[src], idx_r.at[j, 0], sem))
            stage.append(pltpu.async_copy(t_hbm.at[src], idx_t.at[j, 0], sem))
        for c in stage:
            c.wait()

        lane = lax.iota(jnp.int32, 16)

        def chunk_body(c, carry):
            j = c // 4
            off = (c % 4) * _TPC

            vecs = []
            for g in range(2):
                s = pl.ds(off + g * 16, 16)
                vecs.append((idx_h[j, 0, s], idx_r[j, 0, s], idx_t[j, 0, s]))

            copies = []
            for g in range(2):
                vh, vr, vt = vecs[g]
                for k in range(16):
                    slot = g * 16 + k
                    for v, tbl, dst in ((vh, ent_hbm, rows_h),
                                        (vr, rel_hbm, rows_r),
                                        (vt, ent_hbm, rows_t)):
                        copies.append(pltpu.async_copy(
                            tbl.at[v[k] >> 3], dst.at[slot], sem))
            for cp in copies:
                cp.wait()

            for g in range(2):
                vh, vr, vt = vecs[g]
                slotv = g * 16 + lane
                mh, mr, mt = vh & 7, vr & 7, vt & 7
                acc = jnp.zeros((16,), jnp.float32)
                for d in range(HIDDEN_DIM):
                    dv = jnp.full((16,), d, jnp.int32)
                    hv = plsc.load_gather(rows_h, [slotv, mh, dv])
                    rv = plsc.load_gather(rows_r, [slotv, mr, dv])
                    tv = plsc.load_gather(rows_t, [slotv, mt, dv])
                    acc = acc + jnp.abs(hv + rv - tv)
                out_v[pl.ds(c * _TPC + g * 16, 16)] = GAMMA - acc
            return carry

        lax.fori_loop(0, _NPH, chunk_body, 0)

        pltpu.sync_copy(out_v, out_hbm.at[pl.ds(base, _BPW)])

    return kge_score


_KERNEL = _make_kernel()


def kernel(sample, entity_embedding, relation_embedding):
    ent_b = entity_embedding.reshape(125000, 8, HIDDEN_DIM)
    rel_b = relation_embedding.reshape(125000, 8, HIDDEN_DIM)
    scores = _KERNEL(sample[:, 0], sample[:, 1], sample[:, 2], ent_b, rel_b)
    return scores.reshape(BATCH, 1)
